# baseline (speedup 1.0000x reference)
import functools

import jax
import jax.numpy as jnp
from jax import lax
from jax.experimental import pallas as pl
from jax.experimental.pallas import tpu as pltpu
from jax.experimental.pallas import tpu_sc as plsc

_K = 16
_RB = 128
_CC = 1024
_RD = 256
_SQRT_HALF = 0.7071067811865476


def _gelu(v):
    return v * 0.5 * (1.0 + lax.erf(v * _SQRT_HALF))


def _zp_body(x_ref, c_ref, w1x_ref, w1c_ref, b1_ref, z_ref, p_ref):
    cw = jnp.dot(c_ref[0], w1c_ref[...], preferred_element_type=jnp.float32)
    xw = jnp.dot(x_ref[0], w1x_ref[...], preferred_element_type=jnp.float32)
    p_ref[0] = cw
    z_ref[0] = xw + cw + b1_ref[...]


def _colmin(a):
    r = a.shape[0]
    while r > 8:
        r //= 2
        a = jnp.minimum(a[:r], a[r:])
    return jnp.min(a, axis=0)


def _colargmin(d, g):
    r = d.shape[0]
    while r > 1:
        r //= 2
        dlo, dhi = d[:r], d[r:]
        glo, ghi = g[:r], g[r:]
        take = (dhi < dlo) | ((dhi == dlo) & (ghi < glo))
        d = jnp.where(take, dhi, dlo)
        g = jnp.where(take, ghi, glo)
    return d[0], g[0]


def _knn_body(ca_ref, cb_ref, idx_ref, vals_ref, gidx_ref):
    b = pl.program_id(0)
    c = pl.program_id(2)
    nch = pl.num_programs(2)
    cc, rb = _CC, _RB
    n = nch * cc
    ca = ca_ref[0]
    cbt = cb_ref[0]
    sqa = jnp.sum(ca * ca, axis=1, keepdims=True)
    sqb = jnp.sum(cbt * cbt, axis=0)
    dots = lax.dot_general(ca, cbt, (((1,), (0,)), ((), ())),
                           preferred_element_type=jnp.float32)
    d2 = jnp.maximum((sqb[None, :] + sqa) - 2.0 * dots, 1e-12)
    liota = lax.broadcasted_iota(jnp.int32, (cc, rb), 0)
    inf = jnp.float32(jnp.inf)
    goff = c * cc

    d = d2
    ms = []
    lchs = []
    for k in range(_K):
        m, lch = _colargmin(d, liota)
        ms.append(m)
        lchs.append(lch + goff)
        if k < _K - 1:
            d = jnp.where(liota == lch[None, :], inf, d)
    vals_ref[pl.ds(c * _K, _K), :] = jnp.stack(ms, axis=0)
    gidx_ref[pl.ds(c * _K, _K), :] = jnp.stack(lchs, axis=0)

    @pl.when(c == nch - 1)
    def _phase2():
        base = b * n
        v = vals_ref[...]
        g = gidx_ref[...]
        for k in range(_K):
            m, chg = _colargmin(v, g)
            idx_ref[0, k, :] = chg + base
            if k < _K - 1:
                v = jnp.where(g == chg[None, :], inf, v)


def _mlp_body(zg_ref, p_ref, w2_ref, b2_ref, w3_ref, b3_ref, bias_ref, out_ref):
    _, kk, rd, hid = zg_ref.shape
    zg = zg_ref[0]
    h1 = _gelu(zg - p_ref[0][None])
    h2 = _gelu(jnp.dot(h1.reshape(kk * rd, hid), w2_ref[...],
                       preferred_element_type=jnp.float32) + b2_ref[...])
    hm = jnp.mean(h2.reshape(kk, rd, hid), axis=0)
    out_ref[0] = (jnp.dot(hm, w3_ref[...], preferred_element_type=jnp.float32)
                  + b3_ref[...] + bias_ref[...])


def _sc_gather(z2d, idxf):
    rt = idxf.shape[0]
    hid = z2d.shape[1]
    mesh = plsc.VectorSubcoreMesh(core_axis_name="c", subcore_axis_name="s")
    nw = mesh.num_cores * mesh.num_subcores
    per_w = rt // nw
    ch = 128
    chunks = per_w // ch

    @functools.partial(
        pl.kernel,
        out_type=jax.ShapeDtypeStruct((rt, hid), jnp.float32),
        mesh=mesh,
        scratch_types=[
            pltpu.VMEM((ch,), jnp.int32),
            pltpu.VMEM((ch, hid), jnp.float32),
            pltpu.SemaphoreType.DMA,
        ],
    )
    def gk(z_hbm, idx_hbm, out_hbm, idx_v, rows_v, sem):
        wid = lax.axis_index("s") * mesh.num_cores + lax.axis_index("c")
        base = wid * per_w

        def body(i, carry):
            off = base + i * ch
            pltpu.sync_copy(idx_hbm.at[pl.ds(off, ch)], idx_v)
            pltpu.async_copy(z_hbm.at[idx_v], rows_v, sem).wait()
            pltpu.sync_copy(rows_v, out_hbm.at[pl.ds(off, ch)])
            return carry

        lax.fori_loop(0, chunks, body, 0)

    return gk(z2d, idxf)


def kernel(x, coords, W1, b1, W2, b2, W3, b3, bias):
    b, n, cin = x.shape
    dim = coords.shape[-1]
    hid = W1.shape[1]
    cout = W3.shape[1]
    k = _K

    cpad = jnp.pad(coords, ((0, 0), (0, 0), (0, 8 - dim)))
    w1c = jnp.pad(W1[:dim], ((0, 8 - dim), (0, 0)))
    w1x = W1[dim:]

    z, p = pl.pallas_call(
        _zp_body,
        grid=(b,),
        in_specs=[
            pl.BlockSpec((1, n, cin), lambda i: (i, 0, 0)),
            pl.BlockSpec((1, n, 8), lambda i: (i, 0, 0)),
            pl.BlockSpec((cin, hid), lambda i: (0, 0)),
            pl.BlockSpec((8, hid), lambda i: (0, 0)),
            pl.BlockSpec((1, hid), lambda i: (0, 0)),
        ],
        out_specs=[
            pl.BlockSpec((1, n, hid), lambda i: (i, 0, 0)),
            pl.BlockSpec((1, n, hid), lambda i: (i, 0, 0)),
        ],
        out_shape=[
            jax.ShapeDtypeStruct((b, n, hid), jnp.float32),
            jax.ShapeDtypeStruct((b, n, hid), jnp.float32),
        ],
    )(x, cpad, w1x, w1c, b1[None])

    cpadt = jnp.swapaxes(cpad, 1, 2)

    idxg = pl.pallas_call(
        _knn_body,
        grid=(b, n // _RB, n // _CC),
        in_specs=[
            pl.BlockSpec((1, _CC, 8), lambda i, j, c: (i, c, 0)),
            pl.BlockSpec((1, 8, _RB), lambda i, j, c: (i, 0, j)),
        ],
        out_specs=pl.BlockSpec((1, k, _RB), lambda i, j, c: (i, 0, j)),
        out_shape=jax.ShapeDtypeStruct((b, k, n), jnp.int32),
        scratch_shapes=[
            pltpu.VMEM(((n // _CC) * k, _RB), jnp.float32),
            pltpu.VMEM(((n // _CC) * k, _RB), jnp.int32),
        ],
    )(cpad, cpadt)

    zg = _sc_gather(z.reshape(b * n, hid), idxg.reshape(b * k * n))

    out = pl.pallas_call(
        _mlp_body,
        grid=(b, n // _RD),
        in_specs=[
            pl.BlockSpec((1, k, _RD, hid), lambda i, j: (i, 0, j, 0)),
            pl.BlockSpec((1, _RD, hid), lambda i, j: (i, j, 0)),
            pl.BlockSpec((hid, hid), lambda i, j: (0, 0)),
            pl.BlockSpec((1, hid), lambda i, j: (0, 0)),
            pl.BlockSpec((hid, cout), lambda i, j: (0, 0)),
            pl.BlockSpec((1, cout), lambda i, j: (0, 0)),
            pl.BlockSpec((1, cout), lambda i, j: (0, 0)),
        ],
        out_specs=pl.BlockSpec((1, _RD, cout), lambda i, j: (i, j, 0)),
        out_shape=jax.ShapeDtypeStruct((b, n, cout), jnp.float32),
    )(zg.reshape(b, k, n, hid), p, W2, b2[None], W3, b3[None], bias[None])

    return out


# speedup vs baseline: 1.2329x; 1.2329x over previous
import functools

import jax
import jax.numpy as jnp
from jax import lax
from jax.experimental import pallas as pl
from jax.experimental.pallas import tpu as pltpu
from jax.experimental.pallas import tpu_sc as plsc

_K = 16
_RB = 128
_CC = 1024
_RD = 256
_SQRT_HALF = 0.7071067811865476


def _gelu(v):
    return v * 0.5 * (1.0 + lax.erf(v * _SQRT_HALF))


def _zp_body(x_ref, c_ref, w1x_ref, w1c_ref, b1_ref, z_ref, p_ref):
    cw = jnp.dot(c_ref[0], w1c_ref[...], preferred_element_type=jnp.float32)
    xw = jnp.dot(x_ref[0], w1x_ref[...], preferred_element_type=jnp.float32)
    p_ref[0] = cw
    z_ref[0] = xw + cw + b1_ref[...]


def _colmin(a):
    r = a.shape[0]
    while r > 8:
        r //= 2
        a = jnp.minimum(a[:r], a[r:])
    return jnp.min(a, axis=0)


def _colargmin(d, g):
    r = d.shape[0]
    while r > 1:
        r //= 2
        dlo, dhi = d[:r], d[r:]
        glo, ghi = g[:r], g[r:]
        take = (dhi < dlo) | ((dhi == dlo) & (ghi < glo))
        d = jnp.where(take, dhi, dlo)
        g = jnp.where(take, ghi, glo)
    return d[0], g[0]


def _knn_body(ca_ref, cb_ref, idx_ref, vals_ref, gidx_ref):
    b = pl.program_id(0)
    c = pl.program_id(2)
    nch = pl.num_programs(2)
    cc, rb = _CC, _RB
    n = nch * cc
    ca = ca_ref[0]
    cbt = cb_ref[0]
    sqa = jnp.sum(ca * ca, axis=1, keepdims=True)
    sqb = jnp.sum(cbt * cbt, axis=0)
    dots = lax.dot_general(ca, cbt, (((1,), (0,)), ((), ())),
                           preferred_element_type=jnp.float32)
    d2 = jnp.maximum((sqb[None, :] + sqa) - 2.0 * dots, 1e-12)
    liota = lax.broadcasted_iota(jnp.int32, (cc, rb), 0)
    inf = jnp.float32(jnp.inf)
    goff = c * cc

    d = d2
    ms = []
    lchs = []
    for k in range(_K):
        m = _colmin(d)
        cand = jnp.where(d == m[None, :], liota, cc)
        lch = _colmin(cand)
        ms.append(m)
        lchs.append(lch + goff)
        if k < _K - 1:
            d = jnp.where(liota == lch[None, :], inf, d)
    vals_ref[pl.ds(c * _K, _K), :] = jnp.stack(ms, axis=0)
    gidx_ref[pl.ds(c * _K, _K), :] = jnp.stack(lchs, axis=0)

    @pl.when(c == nch - 1)
    def _phase2():
        base = b * n
        v = vals_ref[...]
        g = gidx_ref[...]
        for k in range(_K):
            m = _colmin(v)
            cand2 = jnp.where(v == m[None, :], g, n)
            chg = _colmin(cand2)
            idx_ref[0, k, :] = chg + base
            if k < _K - 1:
                v = jnp.where(g == chg[None, :], inf, v)


def _mlp_body(zg_ref, p_ref, w2_ref, b2_ref, w3_ref, b3_ref, bias_ref, out_ref):
    _, kk, rd, hid = zg_ref.shape
    zg = zg_ref[0]
    h1 = _gelu(zg - p_ref[0][None])
    h2 = _gelu(jnp.dot(h1.reshape(kk * rd, hid), w2_ref[...],
                       preferred_element_type=jnp.float32) + b2_ref[...])
    hm = jnp.mean(h2.reshape(kk, rd, hid), axis=0)
    out_ref[0] = (jnp.dot(hm, w3_ref[...], preferred_element_type=jnp.float32)
                  + b3_ref[...] + bias_ref[...])


def _sc_gather(z2d, idxf):
    rt = idxf.shape[0]
    hid = z2d.shape[1]
    mesh = plsc.VectorSubcoreMesh(core_axis_name="c", subcore_axis_name="s")
    nw = mesh.num_cores * mesh.num_subcores
    per_w = rt // nw
    ch = 128
    chunks = per_w // ch

    nbuf = 4
    assert chunks % nbuf == 0

    @functools.partial(
        pl.kernel,
        out_type=jax.ShapeDtypeStruct((rt, hid), jnp.float32),
        mesh=mesh,
        scratch_types=[
            [pltpu.VMEM((ch,), jnp.int32) for _ in range(nbuf)],
            [pltpu.VMEM((ch, hid), jnp.float32) for _ in range(nbuf)],
            [pltpu.SemaphoreType.DMA for _ in range(nbuf)],
            [pltpu.SemaphoreType.DMA for _ in range(nbuf)],
        ],
    )
    def gk(z_hbm, idx_hbm, out_hbm, idx_vs, rows_vs, gsems, osems):
        wid = lax.axis_index("s") * mesh.num_cores + lax.axis_index("c")
        base = wid * per_w

        def body(i, carry):
            gcps = []
            for j in range(nbuf):
                off = base + (i * nbuf + j) * ch
                pltpu.sync_copy(idx_hbm.at[pl.ds(off, ch)], idx_vs[j])
                gcps.append(pltpu.async_copy(z_hbm.at[idx_vs[j]], rows_vs[j],
                                             gsems[j]))
            ocps = []
            for j in range(nbuf):
                off = base + (i * nbuf + j) * ch
                gcps[j].wait()
                ocps.append(pltpu.async_copy(rows_vs[j],
                                             out_hbm.at[pl.ds(off, ch)],
                                             osems[j]))
            for j in range(nbuf):
                ocps[j].wait()
            return carry

        lax.fori_loop(0, chunks // nbuf, body, 0)

    return gk(z2d, idxf)


def kernel(x, coords, W1, b1, W2, b2, W3, b3, bias):
    b, n, cin = x.shape
    dim = coords.shape[-1]
    hid = W1.shape[1]
    cout = W3.shape[1]
    k = _K

    cpad = jnp.pad(coords, ((0, 0), (0, 0), (0, 8 - dim)))
    w1c = jnp.pad(W1[:dim], ((0, 8 - dim), (0, 0)))
    w1x = W1[dim:]

    z, p = pl.pallas_call(
        _zp_body,
        grid=(b,),
        in_specs=[
            pl.BlockSpec((1, n, cin), lambda i: (i, 0, 0)),
            pl.BlockSpec((1, n, 8), lambda i: (i, 0, 0)),
            pl.BlockSpec((cin, hid), lambda i: (0, 0)),
            pl.BlockSpec((8, hid), lambda i: (0, 0)),
            pl.BlockSpec((1, hid), lambda i: (0, 0)),
        ],
        out_specs=[
            pl.BlockSpec((1, n, hid), lambda i: (i, 0, 0)),
            pl.BlockSpec((1, n, hid), lambda i: (i, 0, 0)),
        ],
        out_shape=[
            jax.ShapeDtypeStruct((b, n, hid), jnp.float32),
            jax.ShapeDtypeStruct((b, n, hid), jnp.float32),
        ],
    )(x, cpad, w1x, w1c, b1[None])

    cpadt = jnp.swapaxes(cpad, 1, 2)

    idxg = pl.pallas_call(
        _knn_body,
        grid=(b, n // _RB, n // _CC),
        in_specs=[
            pl.BlockSpec((1, _CC, 8), lambda i, j, c: (i, c, 0)),
            pl.BlockSpec((1, 8, _RB), lambda i, j, c: (i, 0, j)),
        ],
        out_specs=pl.BlockSpec((1, k, _RB), lambda i, j, c: (i, 0, j)),
        out_shape=jax.ShapeDtypeStruct((b, k, n), jnp.int32),
        scratch_shapes=[
            pltpu.VMEM(((n // _CC) * k, _RB), jnp.float32),
            pltpu.VMEM(((n // _CC) * k, _RB), jnp.int32),
        ],
    )(cpad, cpadt)

    zg = _sc_gather(z.reshape(b * n, hid), idxg.reshape(b * k * n))

    out = pl.pallas_call(
        _mlp_body,
        grid=(b, n // _RD),
        in_specs=[
            pl.BlockSpec((1, k, _RD, hid), lambda i, j: (i, 0, j, 0)),
            pl.BlockSpec((1, _RD, hid), lambda i, j: (i, j, 0)),
            pl.BlockSpec((hid, hid), lambda i, j: (0, 0)),
            pl.BlockSpec((1, hid), lambda i, j: (0, 0)),
            pl.BlockSpec((hid, cout), lambda i, j: (0, 0)),
            pl.BlockSpec((1, cout), lambda i, j: (0, 0)),
            pl.BlockSpec((1, cout), lambda i, j: (0, 0)),
        ],
        out_specs=pl.BlockSpec((1, _RD, cout), lambda i, j: (i, j, 0)),
        out_shape=jax.ShapeDtypeStruct((b, n, cout), jnp.float32),
    )(zg.reshape(b, k, n, hid), p, W2, b2[None], W3, b3[None], bias[None])

    return out


# speedup vs baseline: 1.2649x; 1.0260x over previous
import functools

import jax
import jax.numpy as jnp
from jax import lax
from jax.experimental import pallas as pl
from jax.experimental.pallas import tpu as pltpu
from jax.experimental.pallas import tpu_sc as plsc

_K = 16
_RB = 128
_CC = 1024
_RD = 256
_SQRT_HALF = 0.7071067811865476


def _gelu(v):
    return v * 0.5 * (1.0 + lax.erf(v * _SQRT_HALF))


def _zp_body(x_ref, c_ref, w1x_ref, w1c_ref, b1_ref, z_ref, p_ref):
    cw = jnp.dot(c_ref[0], w1c_ref[...], preferred_element_type=jnp.float32)
    xw = jnp.dot(x_ref[0], w1x_ref[...], preferred_element_type=jnp.float32)
    p_ref[0] = cw
    z_ref[0] = xw + cw + b1_ref[...]


def _colmin(a):
    r = a.shape[0]
    while r > 8:
        r //= 2
        a = jnp.minimum(a[:r], a[r:])
    return jnp.min(a, axis=0)


def _colargmin(d, g):
    r = d.shape[0]
    while r > 1:
        r //= 2
        dlo, dhi = d[:r], d[r:]
        glo, ghi = g[:r], g[r:]
        take = (dhi < dlo) | ((dhi == dlo) & (ghi < glo))
        d = jnp.where(take, dhi, dlo)
        g = jnp.where(take, ghi, glo)
    return d[0], g[0]


def _knn_body(ca_ref, cb_ref, idx_ref, vals_ref, gidx_ref):
    b = pl.program_id(0)
    c = pl.program_id(2)
    nch = pl.num_programs(2)
    cc, rb = _CC, _RB
    n = nch * cc
    ca = ca_ref[0]
    cbt = cb_ref[0]
    sqa = jnp.sum(ca * ca, axis=1, keepdims=True)
    sqb = jnp.sum(cbt * cbt, axis=0)
    dots = lax.dot_general(ca, cbt, (((1,), (0,)), ((), ())),
                           preferred_element_type=jnp.float32)
    d2 = jnp.maximum((sqb[None, :] + sqa) - 2.0 * dots, 1e-12)
    liota = lax.broadcasted_iota(jnp.int32, (cc, rb), 0)
    inf = jnp.float32(jnp.inf)
    goff = c * cc

    d = d2
    ms = []
    lchs = []
    for k in range(_K):
        m = _colmin(d)
        cand = jnp.where(d == m[None, :], liota, cc)
        lch = _colmin(cand)
        ms.append(m)
        lchs.append(lch + goff)
        if k < _K - 1:
            d = jnp.where(liota == lch[None, :], inf, d)
    vals_ref[pl.ds(c * _K, _K), :] = jnp.stack(ms, axis=0)
    gidx_ref[pl.ds(c * _K, _K), :] = jnp.stack(lchs, axis=0)

    @pl.when(c == nch - 1)
    def _phase2():
        base = b * n
        v = vals_ref[...]
        g = gidx_ref[...]
        for k in range(_K):
            m = _colmin(v)
            cand2 = jnp.where(v == m[None, :], g, n)
            chg = _colmin(cand2)
            idx_ref[0, k, :] = chg + base
            if k < _K - 1:
                v = jnp.where(g == chg[None, :], inf, v)


def _mlp_body(zg_ref, p_ref, w2_ref, b2_ref, w3_ref, b3_ref, bias_ref, out_ref):
    _, kk, rd, hid = zg_ref.shape
    zg = zg_ref[0]
    h1 = _gelu(zg - p_ref[0][None])
    h2 = _gelu(jnp.dot(h1.reshape(kk * rd, hid), w2_ref[...],
                       preferred_element_type=jnp.float32) + b2_ref[...])
    hm = jnp.mean(h2.reshape(kk, rd, hid), axis=0)
    out_ref[0] = (jnp.dot(hm, w3_ref[...], preferred_element_type=jnp.float32)
                  + b3_ref[...] + bias_ref[...])


def _sc_gather(z2d, idxf):
    rt = idxf.shape[0]
    hid = z2d.shape[1]
    mesh = plsc.VectorSubcoreMesh(core_axis_name="c", subcore_axis_name="s")
    nw = mesh.num_cores * mesh.num_subcores
    per_w = rt // nw
    ch = 128
    chunks = per_w // ch

    nbuf = 4
    assert chunks % nbuf == 0

    @functools.partial(
        pl.kernel,
        out_type=jax.ShapeDtypeStruct((rt, hid), jnp.float32),
        mesh=mesh,
        scratch_types=[
            [pltpu.VMEM((ch,), jnp.int32) for _ in range(nbuf)],
            [pltpu.VMEM((ch, hid), jnp.float32) for _ in range(nbuf)],
            [pltpu.SemaphoreType.DMA for _ in range(nbuf)],
            [pltpu.SemaphoreType.DMA for _ in range(nbuf)],
        ],
    )
    def gk(z_hbm, idx_hbm, out_hbm, idx_vs, rows_vs, gsems, osems):
        wid = lax.axis_index("s") * mesh.num_cores + lax.axis_index("c")
        base = wid * per_w

        def body(i, carry):
            gcps = []
            for j in range(nbuf):
                off = base + (i * nbuf + j) * ch
                pltpu.sync_copy(idx_hbm.at[pl.ds(off, ch)], idx_vs[j])
                gcps.append(pltpu.async_copy(z_hbm.at[idx_vs[j]], rows_vs[j],
                                             gsems[j]))
            ocps = []
            for j in range(nbuf):
                off = base + (i * nbuf + j) * ch
                gcps[j].wait()
                ocps.append(pltpu.async_copy(rows_vs[j],
                                             out_hbm.at[pl.ds(off, ch)],
                                             osems[j]))
            for j in range(nbuf):
                ocps[j].wait()
            return carry

        lax.fori_loop(0, chunks // nbuf, body, 0)

    return gk(z2d, idxf)


def kernel(x, coords, W1, b1, W2, b2, W3, b3, bias):
    b, n, cin = x.shape
    dim = coords.shape[-1]
    hid = W1.shape[1]
    cout = W3.shape[1]
    k = _K

    cpad = jnp.pad(coords, ((0, 0), (0, 0), (0, 8 - dim)))
    w1c = jnp.pad(W1[:dim], ((0, 8 - dim), (0, 0)))
    w1x = W1[dim:]

    z, p = pl.pallas_call(
        _zp_body,
        grid=(b,),
        in_specs=[
            pl.BlockSpec((1, n, cin), lambda i: (i, 0, 0)),
            pl.BlockSpec((1, n, 8), lambda i: (i, 0, 0)),
            pl.BlockSpec((cin, hid), lambda i: (0, 0)),
            pl.BlockSpec((8, hid), lambda i: (0, 0)),
            pl.BlockSpec((1, hid), lambda i: (0, 0)),
        ],
        out_specs=[
            pl.BlockSpec((1, n, hid), lambda i: (i, 0, 0)),
            pl.BlockSpec((1, n, hid), lambda i: (i, 0, 0)),
        ],
        out_shape=[
            jax.ShapeDtypeStruct((b, n, hid), jnp.float32),
            jax.ShapeDtypeStruct((b, n, hid), jnp.float32),
        ],
    )(x, cpad, w1x, w1c, b1[None])

    cpadt = jnp.swapaxes(cpad, 1, 2)

    outs = []
    for bi in range(b):
        idx_b = pl.pallas_call(
            _knn_body,
            grid=(1, n // _RB, n // _CC),
            in_specs=[
                pl.BlockSpec((1, _CC, 8), lambda i, j, c: (i, c, 0)),
                pl.BlockSpec((1, 8, _RB), lambda i, j, c: (i, 0, j)),
            ],
            out_specs=pl.BlockSpec((1, k, _RB), lambda i, j, c: (i, 0, j)),
            out_shape=jax.ShapeDtypeStruct((1, k, n), jnp.int32),
            scratch_shapes=[
                pltpu.VMEM(((n // _CC) * k, _RB), jnp.float32),
                pltpu.VMEM(((n // _CC) * k, _RB), jnp.int32),
            ],
        )(cpad[bi:bi + 1], cpadt[bi:bi + 1])

        zg_b = _sc_gather(z[bi], idx_b.reshape(k * n))

        out_b = pl.pallas_call(
            _mlp_body,
            grid=(1, n // _RD),
            in_specs=[
                pl.BlockSpec((1, k, _RD, hid), lambda i, j: (i, 0, j, 0)),
                pl.BlockSpec((1, _RD, hid), lambda i, j: (i, j, 0)),
                pl.BlockSpec((hid, hid), lambda i, j: (0, 0)),
                pl.BlockSpec((1, hid), lambda i, j: (0, 0)),
                pl.BlockSpec((hid, cout), lambda i, j: (0, 0)),
                pl.BlockSpec((1, cout), lambda i, j: (0, 0)),
                pl.BlockSpec((1, cout), lambda i, j: (0, 0)),
            ],
            out_specs=pl.BlockSpec((1, _RD, cout), lambda i, j: (i, j, 0)),
            out_shape=jax.ShapeDtypeStruct((1, n, cout), jnp.float32),
        )(zg_b.reshape(1, k, n, hid), p[bi:bi + 1], W2, b2[None], W3,
          b3[None], bias[None])
        outs.append(out_b)

    return jnp.concatenate(outs, axis=0)


# speedup vs baseline: 1.2814x; 1.0130x over previous
import functools

import jax
import jax.numpy as jnp
from jax import lax
from jax.experimental import pallas as pl
from jax.experimental.pallas import tpu as pltpu
from jax.experimental.pallas import tpu_sc as plsc

_K = 16
_RB = 256
_CC = 512
_RD = 256
_SQRT_HALF = 0.7071067811865476


def _gelu(v):
    return v * 0.5 * (1.0 + lax.erf(v * _SQRT_HALF))


def _zp_body(x_ref, c_ref, w1x_ref, w1c_ref, b1_ref, z_ref, p_ref):
    cw = jnp.dot(c_ref[0], w1c_ref[...], preferred_element_type=jnp.float32)
    xw = jnp.dot(x_ref[0], w1x_ref[...], preferred_element_type=jnp.float32)
    p_ref[0] = cw
    z_ref[0] = xw + cw + b1_ref[...]


def _colmin(a):
    r = a.shape[0]
    while r > 8:
        r //= 2
        a = jnp.minimum(a[:r], a[r:])
    return jnp.min(a, axis=0)


def _colargmin(d, g):
    r = d.shape[0]
    while r > 1:
        r //= 2
        dlo, dhi = d[:r], d[r:]
        glo, ghi = g[:r], g[r:]
        take = (dhi < dlo) | ((dhi == dlo) & (ghi < glo))
        d = jnp.where(take, dhi, dlo)
        g = jnp.where(take, ghi, glo)
    return d[0], g[0]


def _knn_body(ca_ref, cb_ref, idx_ref, vals_ref, gidx_ref):
    b = pl.program_id(0)
    c = pl.program_id(2)
    nch = pl.num_programs(2)
    cc, rb = _CC, _RB
    n = nch * cc
    ca = ca_ref[0]
    cbt = cb_ref[0]
    sqa = jnp.sum(ca * ca, axis=1, keepdims=True)
    sqb = jnp.sum(cbt * cbt, axis=0)
    dots = lax.dot_general(ca, cbt, (((1,), (0,)), ((), ())),
                           preferred_element_type=jnp.float32)
    d2 = jnp.maximum((sqb[None, :] + sqa) - 2.0 * dots, 1e-12)
    liota = lax.broadcasted_iota(jnp.int32, (cc, rb), 0)
    inf = jnp.float32(jnp.inf)
    goff = c * cc

    d = d2
    ms = []
    lchs = []
    for k in range(_K):
        m = _colmin(d)
        cand = jnp.where(d == m[None, :], liota, cc)
        lch = _colmin(cand)
        ms.append(m)
        lchs.append(lch + goff)
        if k < _K - 1:
            d = jnp.where(liota == lch[None, :], inf, d)
    vals_ref[pl.ds(c * _K, _K), :] = jnp.stack(ms, axis=0)
    gidx_ref[pl.ds(c * _K, _K), :] = jnp.stack(lchs, axis=0)

    @pl.when(c == nch - 1)
    def _phase2():
        base = b * n
        v = vals_ref[...]
        g = gidx_ref[...]
        for k in range(_K):
            m = _colmin(v)
            cand2 = jnp.where(v == m[None, :], g, n)
            chg = _colmin(cand2)
            idx_ref[0, k, :] = chg + base
            if k < _K - 1:
                v = jnp.where(g == chg[None, :], inf, v)


def _mlp_body(zg_ref, p_ref, w2_ref, b2_ref, w3_ref, b3_ref, bias_ref, out_ref):
    _, kk, rd, hid = zg_ref.shape
    zg = zg_ref[0]
    h1 = _gelu(zg - p_ref[0][None])
    h2 = _gelu(jnp.dot(h1.reshape(kk * rd, hid), w2_ref[...],
                       preferred_element_type=jnp.float32) + b2_ref[...])
    hm = jnp.mean(h2.reshape(kk, rd, hid), axis=0)
    out_ref[0] = (jnp.dot(hm, w3_ref[...], preferred_element_type=jnp.float32)
                  + b3_ref[...] + bias_ref[...])


def _sc_gather(z2d, idxf):
    rt = idxf.shape[0]
    hid = z2d.shape[1]
    mesh = plsc.VectorSubcoreMesh(core_axis_name="c", subcore_axis_name="s")
    nw = mesh.num_cores * mesh.num_subcores
    per_w = rt // nw
    ch = 128
    chunks = per_w // ch

    nbuf = 4
    assert chunks % nbuf == 0

    @functools.partial(
        pl.kernel,
        out_type=jax.ShapeDtypeStruct((rt, hid), jnp.float32),
        mesh=mesh,
        scratch_types=[
            [pltpu.VMEM((ch,), jnp.int32) for _ in range(nbuf)],
            [pltpu.VMEM((ch, hid), jnp.float32) for _ in range(nbuf)],
            [pltpu.SemaphoreType.DMA for _ in range(nbuf)],
            [pltpu.SemaphoreType.DMA for _ in range(nbuf)],
        ],
    )
    def gk(z_hbm, idx_hbm, out_hbm, idx_vs, rows_vs, gsems, osems):
        wid = lax.axis_index("s") * mesh.num_cores + lax.axis_index("c")
        base = wid * per_w

        def body(i, carry):
            gcps = []
            for j in range(nbuf):
                off = base + (i * nbuf + j) * ch
                pltpu.sync_copy(idx_hbm.at[pl.ds(off, ch)], idx_vs[j])
                gcps.append(pltpu.async_copy(z_hbm.at[idx_vs[j]], rows_vs[j],
                                             gsems[j]))
            ocps = []
            for j in range(nbuf):
                off = base + (i * nbuf + j) * ch
                gcps[j].wait()
                ocps.append(pltpu.async_copy(rows_vs[j],
                                             out_hbm.at[pl.ds(off, ch)],
                                             osems[j]))
            for j in range(nbuf):
                ocps[j].wait()
            return carry

        lax.fori_loop(0, chunks // nbuf, body, 0)

    return gk(z2d, idxf)


def kernel(x, coords, W1, b1, W2, b2, W3, b3, bias):
    b, n, cin = x.shape
    dim = coords.shape[-1]
    hid = W1.shape[1]
    cout = W3.shape[1]
    k = _K

    cpad = jnp.pad(coords, ((0, 0), (0, 0), (0, 8 - dim)))
    w1c = jnp.pad(W1[:dim], ((0, 8 - dim), (0, 0)))
    w1x = W1[dim:]

    z, p = pl.pallas_call(
        _zp_body,
        grid=(b,),
        in_specs=[
            pl.BlockSpec((1, n, cin), lambda i: (i, 0, 0)),
            pl.BlockSpec((1, n, 8), lambda i: (i, 0, 0)),
            pl.BlockSpec((cin, hid), lambda i: (0, 0)),
            pl.BlockSpec((8, hid), lambda i: (0, 0)),
            pl.BlockSpec((1, hid), lambda i: (0, 0)),
        ],
        out_specs=[
            pl.BlockSpec((1, n, hid), lambda i: (i, 0, 0)),
            pl.BlockSpec((1, n, hid), lambda i: (i, 0, 0)),
        ],
        out_shape=[
            jax.ShapeDtypeStruct((b, n, hid), jnp.float32),
            jax.ShapeDtypeStruct((b, n, hid), jnp.float32),
        ],
    )(x, cpad, w1x, w1c, b1[None])

    cpadt = jnp.swapaxes(cpad, 1, 2)

    outs = []
    for bi in range(b):
        idx_b = pl.pallas_call(
            _knn_body,
            grid=(1, n // _RB, n // _CC),
            in_specs=[
                pl.BlockSpec((1, _CC, 8), lambda i, j, c: (i, c, 0)),
                pl.BlockSpec((1, 8, _RB), lambda i, j, c: (i, 0, j)),
            ],
            out_specs=pl.BlockSpec((1, k, _RB), lambda i, j, c: (i, 0, j)),
            out_shape=jax.ShapeDtypeStruct((1, k, n), jnp.int32),
            scratch_shapes=[
                pltpu.VMEM(((n // _CC) * k, _RB), jnp.float32),
                pltpu.VMEM(((n // _CC) * k, _RB), jnp.int32),
            ],
        )(cpad[bi:bi + 1], cpadt[bi:bi + 1])

        zg_b = _sc_gather(z[bi], idx_b.reshape(k * n))

        out_b = pl.pallas_call(
            _mlp_body,
            grid=(1, n // _RD),
            in_specs=[
                pl.BlockSpec((1, k, _RD, hid), lambda i, j: (i, 0, j, 0)),
                pl.BlockSpec((1, _RD, hid), lambda i, j: (i, j, 0)),
                pl.BlockSpec((hid, hid), lambda i, j: (0, 0)),
                pl.BlockSpec((1, hid), lambda i, j: (0, 0)),
                pl.BlockSpec((hid, cout), lambda i, j: (0, 0)),
                pl.BlockSpec((1, cout), lambda i, j: (0, 0)),
                pl.BlockSpec((1, cout), lambda i, j: (0, 0)),
            ],
            out_specs=pl.BlockSpec((1, _RD, cout), lambda i, j: (i, j, 0)),
            out_shape=jax.ShapeDtypeStruct((1, n, cout), jnp.float32),
        )(zg_b.reshape(1, k, n, hid), p[bi:bi + 1], W2, b2[None], W3,
          b3[None], bias[None])
        outs.append(out_b)

    return jnp.concatenate(outs, axis=0)


# speedup vs baseline: 1.2911x; 1.0076x over previous
import functools

import jax
import jax.numpy as jnp
from jax import lax
from jax.experimental import pallas as pl
from jax.experimental.pallas import tpu as pltpu
from jax.experimental.pallas import tpu_sc as plsc

_K = 16
_RB = 256
_CC = 512
_RD = 512
_SQRT_HALF = 0.7071067811865476


def _gelu(v):
    return v * 0.5 * (1.0 + lax.erf(v * _SQRT_HALF))


def _zp_body(x_ref, c_ref, w1x_ref, w1c_ref, b1_ref, z_ref, p_ref):
    cw = jnp.dot(c_ref[0], w1c_ref[...], preferred_element_type=jnp.float32)
    xw = jnp.dot(x_ref[0], w1x_ref[...], preferred_element_type=jnp.float32)
    p_ref[0] = cw
    z_ref[0] = xw + cw + b1_ref[...]


def _colmin(a):
    r = a.shape[0]
    while r > 8:
        r //= 2
        a = jnp.minimum(a[:r], a[r:])
    return jnp.min(a, axis=0)


def _colargmin(d, g):
    r = d.shape[0]
    while r > 1:
        r //= 2
        dlo, dhi = d[:r], d[r:]
        glo, ghi = g[:r], g[r:]
        take = (dhi < dlo) | ((dhi == dlo) & (ghi < glo))
        d = jnp.where(take, dhi, dlo)
        g = jnp.where(take, ghi, glo)
    return d[0], g[0]


def _knn_body(ca_ref, cb_ref, idx_ref, vals_ref, gidx_ref):
    b = pl.program_id(0)
    c = pl.program_id(2)
    nch = pl.num_programs(2)
    cc, rb = _CC, _RB
    n = nch * cc
    ca = ca_ref[0]
    cbt = cb_ref[0]
    sqa = jnp.sum(ca * ca, axis=1, keepdims=True)
    sqb = jnp.sum(cbt * cbt, axis=0)
    dots = lax.dot_general(ca, cbt, (((1,), (0,)), ((), ())),
                           preferred_element_type=jnp.float32)
    d2 = jnp.maximum((sqb[None, :] + sqa) - 2.0 * dots, 1e-12)
    liota = lax.broadcasted_iota(jnp.int32, (cc, rb), 0)
    inf = jnp.float32(jnp.inf)
    goff = c * cc

    d = d2
    ms = []
    lchs = []
    for k in range(_K):
        m = _colmin(d)
        cand = jnp.where(d == m[None, :], liota, cc)
        lch = _colmin(cand)
        ms.append(m)
        lchs.append(lch + goff)
        if k < _K - 1:
            d = jnp.where(liota == lch[None, :], inf, d)
    vals_ref[pl.ds(c * _K, _K), :] = jnp.stack(ms, axis=0)
    gidx_ref[pl.ds(c * _K, _K), :] = jnp.stack(lchs, axis=0)

    @pl.when(c == nch - 1)
    def _phase2():
        base = b * n
        v = vals_ref[...]
        g = gidx_ref[...]
        for k in range(_K):
            m = _colmin(v)
            cand2 = jnp.where(v == m[None, :], g, n)
            chg = _colmin(cand2)
            idx_ref[0, k, :] = chg + base
            if k < _K - 1:
                v = jnp.where(g == chg[None, :], inf, v)


def _mlp_body(zg_ref, p_ref, w2_ref, b2_ref, w3_ref, b3_ref, bias_ref, out_ref):
    _, kk, rd, hid = zg_ref.shape
    zg = zg_ref[0]
    h1 = _gelu(zg - p_ref[0][None])
    h2 = _gelu(jnp.dot(h1.reshape(kk * rd, hid), w2_ref[...],
                       preferred_element_type=jnp.float32) + b2_ref[...])
    hm = jnp.mean(h2.reshape(kk, rd, hid), axis=0)
    out_ref[0] = (jnp.dot(hm, w3_ref[...], preferred_element_type=jnp.float32)
                  + b3_ref[...] + bias_ref[...])


def _sc_gather(z2d, idxf):
    rt = idxf.shape[0]
    hid = z2d.shape[1]
    mesh = plsc.VectorSubcoreMesh(core_axis_name="c", subcore_axis_name="s")
    nw = mesh.num_cores * mesh.num_subcores
    per_w = rt // nw
    ch = 128
    chunks = per_w // ch

    nbuf = 4
    assert chunks % nbuf == 0

    @functools.partial(
        pl.kernel,
        out_type=jax.ShapeDtypeStruct((rt, hid), jnp.float32),
        mesh=mesh,
        scratch_types=[
            [pltpu.VMEM((ch,), jnp.int32) for _ in range(nbuf)],
            [pltpu.VMEM((ch, hid), jnp.float32) for _ in range(nbuf)],
            [pltpu.SemaphoreType.DMA for _ in range(nbuf)],
            [pltpu.SemaphoreType.DMA for _ in range(nbuf)],
        ],
    )
    def gk(z_hbm, idx_hbm, out_hbm, idx_vs, rows_vs, gsems, osems):
        wid = lax.axis_index("s") * mesh.num_cores + lax.axis_index("c")
        base = wid * per_w

        def body(i, carry):
            gcps = []
            for j in range(nbuf):
                off = base + (i * nbuf + j) * ch
                pltpu.sync_copy(idx_hbm.at[pl.ds(off, ch)], idx_vs[j])
                gcps.append(pltpu.async_copy(z_hbm.at[idx_vs[j]], rows_vs[j],
                                             gsems[j]))
            ocps = []
            for j in range(nbuf):
                off = base + (i * nbuf + j) * ch
                gcps[j].wait()
                ocps.append(pltpu.async_copy(rows_vs[j],
                                             out_hbm.at[pl.ds(off, ch)],
                                             osems[j]))
            for j in range(nbuf):
                ocps[j].wait()
            return carry

        lax.fori_loop(0, chunks // nbuf, body, 0)

    return gk(z2d, idxf)


def kernel(x, coords, W1, b1, W2, b2, W3, b3, bias):
    b, n, cin = x.shape
    dim = coords.shape[-1]
    hid = W1.shape[1]
    cout = W3.shape[1]
    k = _K

    cpad = jnp.pad(coords, ((0, 0), (0, 0), (0, 8 - dim)))
    w1c = jnp.pad(W1[:dim], ((0, 8 - dim), (0, 0)))
    w1x = W1[dim:]

    z, p = pl.pallas_call(
        _zp_body,
        grid=(b,),
        in_specs=[
            pl.BlockSpec((1, n, cin), lambda i: (i, 0, 0)),
            pl.BlockSpec((1, n, 8), lambda i: (i, 0, 0)),
            pl.BlockSpec((cin, hid), lambda i: (0, 0)),
            pl.BlockSpec((8, hid), lambda i: (0, 0)),
            pl.BlockSpec((1, hid), lambda i: (0, 0)),
        ],
        out_specs=[
            pl.BlockSpec((1, n, hid), lambda i: (i, 0, 0)),
            pl.BlockSpec((1, n, hid), lambda i: (i, 0, 0)),
        ],
        out_shape=[
            jax.ShapeDtypeStruct((b, n, hid), jnp.float32),
            jax.ShapeDtypeStruct((b, n, hid), jnp.float32),
        ],
    )(x, cpad, w1x, w1c, b1[None])

    cpadt = jnp.swapaxes(cpad, 1, 2)

    outs = []
    for bi in range(b):
        idx_b = pl.pallas_call(
            _knn_body,
            grid=(1, n // _RB, n // _CC),
            in_specs=[
                pl.BlockSpec((1, _CC, 8), lambda i, j, c: (i, c, 0)),
                pl.BlockSpec((1, 8, _RB), lambda i, j, c: (i, 0, j)),
            ],
            out_specs=pl.BlockSpec((1, k, _RB), lambda i, j, c: (i, 0, j)),
            out_shape=jax.ShapeDtypeStruct((1, k, n), jnp.int32),
            scratch_shapes=[
                pltpu.VMEM(((n // _CC) * k, _RB), jnp.float32),
                pltpu.VMEM(((n // _CC) * k, _RB), jnp.int32),
            ],
        )(cpad[bi:bi + 1], cpadt[bi:bi + 1])

        zg_b = _sc_gather(z[bi], idx_b.reshape(k * n))

        out_b = pl.pallas_call(
            _mlp_body,
            grid=(1, n // _RD),
            in_specs=[
                pl.BlockSpec((1, k, _RD, hid), lambda i, j: (i, 0, j, 0)),
                pl.BlockSpec((1, _RD, hid), lambda i, j: (i, j, 0)),
                pl.BlockSpec((hid, hid), lambda i, j: (0, 0)),
                pl.BlockSpec((1, hid), lambda i, j: (0, 0)),
                pl.BlockSpec((hid, cout), lambda i, j: (0, 0)),
                pl.BlockSpec((1, cout), lambda i, j: (0, 0)),
                pl.BlockSpec((1, cout), lambda i, j: (0, 0)),
            ],
            out_specs=pl.BlockSpec((1, _RD, cout), lambda i, j: (i, j, 0)),
            out_shape=jax.ShapeDtypeStruct((1, n, cout), jnp.float32),
        )(zg_b.reshape(1, k, n, hid), p[bi:bi + 1], W2, b2[None], W3,
          b3[None], bias[None])
        outs.append(out_b)

    return jnp.concatenate(outs, axis=0)


# speedup vs baseline: 1.2982x; 1.0055x over previous
import functools

import jax
import jax.numpy as jnp
from jax import lax
from jax.experimental import pallas as pl
from jax.experimental.pallas import tpu as pltpu
from jax.experimental.pallas import tpu_sc as plsc

_K = 16
_RB = 256
_CC = 512
_RD = 512
_SQRT_HALF = 0.7071067811865476


def _gelu(v):
    return v * 0.5 * (1.0 + lax.erf(v * _SQRT_HALF))


def _zp_body(x_ref, c_ref, w1x_ref, w1c_ref, b1_ref, z_ref, p_ref):
    cw = jnp.dot(c_ref[0], w1c_ref[...], preferred_element_type=jnp.float32)
    xw = jnp.dot(x_ref[0], w1x_ref[...], preferred_element_type=jnp.float32)
    p_ref[0] = cw
    z_ref[0] = xw + cw + b1_ref[...]


def _colmin(a):
    r = a.shape[0]
    while r > 8:
        r //= 2
        a = jnp.minimum(a[:r], a[r:])
    return jnp.min(a, axis=0)


def _colmin8(a):
    g = a.shape[0]
    while g > 1:
        g //= 2
        a = jnp.minimum(a[:g], a[g:])
    a = a[0]
    for s in (4, 2, 1):
        a = jnp.minimum(a, pltpu.roll(a, s, axis=0))
    return a


def _colargmin(d, g):
    r = d.shape[0]
    while r > 1:
        r //= 2
        dlo, dhi = d[:r], d[r:]
        glo, ghi = g[:r], g[r:]
        take = (dhi < dlo) | ((dhi == dlo) & (ghi < glo))
        d = jnp.where(take, dhi, dlo)
        g = jnp.where(take, ghi, glo)
    return d[0], g[0]


def _knn_body(ca_ref, cb_ref, idx_ref, vals_ref, gidx_ref):
    b = pl.program_id(0)
    c = pl.program_id(2)
    nch = pl.num_programs(2)
    cc, rb = _CC, _RB
    n = nch * cc
    ca = ca_ref[0]
    cbt = cb_ref[0]
    sqa = jnp.sum(ca * ca, axis=1, keepdims=True)
    sqb = jnp.sum(cbt * cbt, axis=0)
    dots = lax.dot_general(ca, cbt, (((1,), (0,)), ((), ())),
                           preferred_element_type=jnp.float32)
    d2 = jnp.maximum((sqb[None, :] + sqa) - 2.0 * dots, 1e-12)
    inf = jnp.float32(jnp.inf)
    goff = c * cc

    d = d2.reshape(cc // 8, 8, rb)
    liota3 = (lax.broadcasted_iota(jnp.int32, (cc // 8, 8, rb), 0) * 8
              + lax.broadcasted_iota(jnp.int32, (cc // 8, 8, rb), 1))
    ms = []
    lchs = []
    for k in range(_K):
        m8 = _colmin8(d)
        cand = jnp.where(d == m8[None], liota3, cc)
        l8 = _colmin8(cand)
        ms.append(m8[0])
        lchs.append(l8[0] + goff)
        if k < _K - 1:
            d = jnp.where(liota3 == l8[None], inf, d)
    vals_ref[pl.ds(c * _K, _K), :] = jnp.stack(ms, axis=0)
    gidx_ref[pl.ds(c * _K, _K), :] = jnp.stack(lchs, axis=0)

    @pl.when(c == nch - 1)
    def _phase2():
        base = b * n
        v = vals_ref[...]
        g = gidx_ref[...]
        for k in range(_K):
            m = _colmin(v)
            cand2 = jnp.where(v == m[None, :], g, n)
            chg = _colmin(cand2)
            idx_ref[0, k, :] = chg + base
            if k < _K - 1:
                v = jnp.where(g == chg[None, :], inf, v)


def _mlp_body(zg_ref, p_ref, w2_ref, b2_ref, w3_ref, b3_ref, bias_ref, out_ref):
    _, kk, rd, hid = zg_ref.shape
    zg = zg_ref[0]
    h1 = _gelu(zg - p_ref[0][None])
    h2 = _gelu(jnp.dot(h1.reshape(kk * rd, hid), w2_ref[...],
                       preferred_element_type=jnp.float32) + b2_ref[...])
    hm = jnp.mean(h2.reshape(kk, rd, hid), axis=0)
    out_ref[0] = (jnp.dot(hm, w3_ref[...], preferred_element_type=jnp.float32)
                  + b3_ref[...] + bias_ref[...])


def _sc_gather(z2d, idxf):
    rt = idxf.shape[0]
    hid = z2d.shape[1]
    mesh = plsc.VectorSubcoreMesh(core_axis_name="c", subcore_axis_name="s")
    nw = mesh.num_cores * mesh.num_subcores
    per_w = rt // nw
    ch = 128
    chunks = per_w // ch

    nbuf = 4
    assert chunks % nbuf == 0

    @functools.partial(
        pl.kernel,
        out_type=jax.ShapeDtypeStruct((rt, hid), jnp.float32),
        mesh=mesh,
        scratch_types=[
            [pltpu.VMEM((ch,), jnp.int32) for _ in range(nbuf)],
            [pltpu.VMEM((ch, hid), jnp.float32) for _ in range(nbuf)],
            [pltpu.SemaphoreType.DMA for _ in range(nbuf)],
            [pltpu.SemaphoreType.DMA for _ in range(nbuf)],
        ],
    )
    def gk(z_hbm, idx_hbm, out_hbm, idx_vs, rows_vs, gsems, osems):
        wid = lax.axis_index("s") * mesh.num_cores + lax.axis_index("c")
        base = wid * per_w

        def body(i, carry):
            gcps = []
            for j in range(nbuf):
                off = base + (i * nbuf + j) * ch
                pltpu.sync_copy(idx_hbm.at[pl.ds(off, ch)], idx_vs[j])
                gcps.append(pltpu.async_copy(z_hbm.at[idx_vs[j]], rows_vs[j],
                                             gsems[j]))
            ocps = []
            for j in range(nbuf):
                off = base + (i * nbuf + j) * ch
                gcps[j].wait()
                ocps.append(pltpu.async_copy(rows_vs[j],
                                             out_hbm.at[pl.ds(off, ch)],
                                             osems[j]))
            for j in range(nbuf):
                ocps[j].wait()
            return carry

        lax.fori_loop(0, chunks // nbuf, body, 0)

    return gk(z2d, idxf)


def kernel(x, coords, W1, b1, W2, b2, W3, b3, bias):
    b, n, cin = x.shape
    dim = coords.shape[-1]
    hid = W1.shape[1]
    cout = W3.shape[1]
    k = _K

    cpad = jnp.pad(coords, ((0, 0), (0, 0), (0, 8 - dim)))
    w1c = jnp.pad(W1[:dim], ((0, 8 - dim), (0, 0)))
    w1x = W1[dim:]

    z, p = pl.pallas_call(
        _zp_body,
        grid=(b,),
        in_specs=[
            pl.BlockSpec((1, n, cin), lambda i: (i, 0, 0)),
            pl.BlockSpec((1, n, 8), lambda i: (i, 0, 0)),
            pl.BlockSpec((cin, hid), lambda i: (0, 0)),
            pl.BlockSpec((8, hid), lambda i: (0, 0)),
            pl.BlockSpec((1, hid), lambda i: (0, 0)),
        ],
        out_specs=[
            pl.BlockSpec((1, n, hid), lambda i: (i, 0, 0)),
            pl.BlockSpec((1, n, hid), lambda i: (i, 0, 0)),
        ],
        out_shape=[
            jax.ShapeDtypeStruct((b, n, hid), jnp.float32),
            jax.ShapeDtypeStruct((b, n, hid), jnp.float32),
        ],
    )(x, cpad, w1x, w1c, b1[None])

    cpadt = jnp.swapaxes(cpad, 1, 2)

    outs = []
    for bi in range(b):
        idx_b = pl.pallas_call(
            _knn_body,
            grid=(1, n // _RB, n // _CC),
            in_specs=[
                pl.BlockSpec((1, _CC, 8), lambda i, j, c: (i, c, 0)),
                pl.BlockSpec((1, 8, _RB), lambda i, j, c: (i, 0, j)),
            ],
            out_specs=pl.BlockSpec((1, k, _RB), lambda i, j, c: (i, 0, j)),
            out_shape=jax.ShapeDtypeStruct((1, k, n), jnp.int32),
            scratch_shapes=[
                pltpu.VMEM(((n // _CC) * k, _RB), jnp.float32),
                pltpu.VMEM(((n // _CC) * k, _RB), jnp.int32),
            ],
        )(cpad[bi:bi + 1], cpadt[bi:bi + 1])

        zg_b = _sc_gather(z[bi], idx_b.reshape(k * n))

        out_b = pl.pallas_call(
            _mlp_body,
            grid=(1, n // _RD),
            in_specs=[
                pl.BlockSpec((1, k, _RD, hid), lambda i, j: (i, 0, j, 0)),
                pl.BlockSpec((1, _RD, hid), lambda i, j: (i, j, 0)),
                pl.BlockSpec((hid, hid), lambda i, j: (0, 0)),
                pl.BlockSpec((1, hid), lambda i, j: (0, 0)),
                pl.BlockSpec((hid, cout), lambda i, j: (0, 0)),
                pl.BlockSpec((1, cout), lambda i, j: (0, 0)),
                pl.BlockSpec((1, cout), lambda i, j: (0, 0)),
            ],
            out_specs=pl.BlockSpec((1, _RD, cout), lambda i, j: (i, j, 0)),
            out_shape=jax.ShapeDtypeStruct((1, n, cout), jnp.float32),
        )(zg_b.reshape(1, k, n, hid), p[bi:bi + 1], W2, b2[None], W3,
          b3[None], bias[None])
        outs.append(out_b)

    return jnp.concatenate(outs, axis=0)
